# baseline (speedup 1.0000x reference)
import functools

import jax
import jax.numpy as jnp
from jax.experimental import pallas as pl

_N_HEADS = 16
_F32 = jnp.float32


def _dot(a, b):
    return jnp.dot(a, b, preferred_element_type=_F32)


def _dot_nt(a, b):
    return jax.lax.dot_general(a, b, (((1,), (1,)), ((), ())),
                               preferred_element_type=_F32)


def _ln_route_body(x_ref, imp_ref, w_ref, g_ref, b_ref,
                   h1_ref, wacc_ref, isum_ref, *, n_groups):
    s = pl.program_id(1)
    xb = x_ref[0]
    mu = jnp.mean(xb, axis=-1, keepdims=True)
    xc = xb - mu
    var = jnp.mean(xc * xc, axis=-1, keepdims=True)
    h1 = xc * jax.lax.rsqrt(var + 1e-5) * g_ref[...] + b_ref[...]
    h1_ref[0] = h1
    logits = _dot(h1, w_ref[...])
    ps = []
    for r in range(n_groups):
        lg = logits[:, r * 64:(r + 1) * 64]
        m = jnp.max(lg, axis=-1, keepdims=True)
        e = jnp.exp(lg - m)
        ps.append(e / jnp.sum(e, axis=-1, keepdims=True))
    p = jnp.concatenate(ps, axis=-1) if n_groups > 1 else ps[0]
    imp = imp_ref[0, 0]

    @pl.when(s == 0)
    def _():
        wacc_ref[...] = jnp.zeros_like(wacc_ref)
        isum_ref[...] = jnp.zeros_like(isum_ref)

    wacc_ref[0] += _dot(imp, p)
    isum_ref[0] += jnp.sum(imp)


def _ln_route(x, importance, w_cat, g, b, n_groups, bs):
    bsz, ssz, d = x.shape
    grid = (bsz, ssz // bs)
    imp4 = importance.reshape(bsz, ssz // bs, 1, bs)
    h1, wacc, isum = pl.pallas_call(
        functools.partial(_ln_route_body, n_groups=n_groups),
        grid=grid,
        in_specs=[
            pl.BlockSpec((1, bs, d), lambda bi, si: (bi, si, 0)),
            pl.BlockSpec((1, 1, 1, bs), lambda bi, si: (bi, si, 0, 0)),
            pl.BlockSpec((d, n_groups * 64), lambda bi, si: (0, 0)),
            pl.BlockSpec((1, d), lambda bi, si: (0, 0)),
            pl.BlockSpec((1, d), lambda bi, si: (0, 0)),
        ],
        out_specs=[
            pl.BlockSpec((1, bs, d), lambda bi, si: (bi, si, 0)),
            pl.BlockSpec((1, 1, n_groups * 64), lambda bi, si: (bi, 0, 0)),
            pl.BlockSpec((1, 1, 128), lambda bi, si: (bi, 0, 0)),
        ],
        out_shape=[
            jax.ShapeDtypeStruct((bsz, ssz, d), _F32),
            jax.ShapeDtypeStruct((bsz, 1, n_groups * 64), _F32),
            jax.ShapeDtypeStruct((bsz, 1, 128), _F32),
        ],
    )(x, imp4, w_cat, g.reshape(1, d), b.reshape(1, d))
    return h1, wacc.reshape(bsz, n_groups * 64), isum.reshape(bsz, 128)


def _topk_body(wacc_ref, isum_ref, wsel_ref, *, ks):
    c = isum_ref[:, :1] + 1e-8
    outs = []
    for r, k in enumerate(ks):
        w = wacc_ref[:, r * 64:(r + 1) * 64] / c
        w = w / (jnp.sum(w, axis=-1, keepdims=True) + 1e-8)
        iota = jax.lax.broadcasted_iota(jnp.int32, w.shape, 1)
        vals = w
        sel = jnp.zeros_like(w)
        for _ in range(k):
            m = jnp.max(vals, axis=-1, keepdims=True)
            idx = jnp.min(jnp.where(vals == m, iota, 64), axis=-1,
                          keepdims=True)
            oh = iota == idx
            sel = jnp.where(oh, w, sel)
            vals = jnp.where(oh, -jnp.inf, vals)
        outs.append(sel / (jnp.sum(sel, axis=-1, keepdims=True) + 1e-8))
    wsel_ref[...] = jnp.concatenate(outs, axis=-1) if len(ks) > 1 else outs[0]


def _topk(wacc, isum, ks):
    bsz, nw = wacc.shape
    return pl.pallas_call(
        functools.partial(_topk_body, ks=ks),
        out_shape=jax.ShapeDtypeStruct((bsz, nw), _F32),
    )(wacc, isum)


def _combine_body(w_ref, pool_ref, out_ref):
    out_ref[...] = _dot(w_ref[...], pool_ref[...])


def _combine(wsel, pool_flat, chunk):
    rows, n = wsel.shape
    _, width = pool_flat.shape
    return pl.pallas_call(
        _combine_body,
        grid=(width // chunk,),
        in_specs=[
            pl.BlockSpec((rows, n), lambda ci: (0, 0)),
            pl.BlockSpec((n, chunk), lambda ci: (0, ci)),
        ],
        out_specs=pl.BlockSpec((rows, chunk), lambda ci: (0, ci)),
        out_shape=jax.ShapeDtypeStruct((rows, width), _F32),
    )(wsel, pool_flat)


def _qkv_body(h1_ref, sc_ref, sq_ref, sk_ref, sv_ref, q_ref, k_ref, v_ref):
    h = _dot(h1_ref[0], sc_ref[0])
    q_ref[0] = _dot(h, sq_ref[0])
    k_ref[0] = _dot(h, sk_ref[0])
    v_ref[0] = _dot(h, sv_ref[0])


def _qkv(h1, shared_c, sq, sk, sv, bs):
    bsz, ssz, d = h1.shape
    r = shared_c.shape[-1]
    big = pl.BlockSpec((1, bs, d), lambda bi, si: (bi, si, 0))
    mat_dr = pl.BlockSpec((1, d, r), lambda bi, si: (bi, 0, 0))
    mat_rd = pl.BlockSpec((1, r, d), lambda bi, si: (bi, 0, 0))
    out = jax.ShapeDtypeStruct((bsz, ssz, d), _F32)
    return pl.pallas_call(
        _qkv_body,
        grid=(bsz, ssz // bs),
        in_specs=[big, mat_dr, mat_rd, mat_rd, mat_rd],
        out_specs=[big, big, big],
        out_shape=[out, out, out],
    )(h1, shared_c, sq, sk, sv)


def _attn_body(q_ref, k_ref, v_ref, o_ref, *, scale):
    q = q_ref[0, 0]
    s = _dot_nt(q, k_ref[0, 0]) * scale
    m = jnp.max(s, axis=-1, keepdims=True)
    p = jnp.exp(s - m)
    p = p / jnp.sum(p, axis=-1, keepdims=True)
    o_ref[0, 0] = _dot(p, v_ref[0, 0])


def _attn(q, k, v, bq, scale):
    bsz, h, ssz, dh = q.shape
    return pl.pallas_call(
        functools.partial(_attn_body, scale=scale),
        grid=(bsz, h, ssz // bq),
        in_specs=[
            pl.BlockSpec((1, 1, bq, dh), lambda bi, hi, qi: (bi, hi, qi, 0)),
            pl.BlockSpec((1, 1, ssz, dh), lambda bi, hi, qi: (bi, hi, 0, 0)),
            pl.BlockSpec((1, 1, ssz, dh), lambda bi, hi, qi: (bi, hi, 0, 0)),
        ],
        out_specs=pl.BlockSpec((1, 1, bq, dh),
                               lambda bi, hi, qi: (bi, hi, qi, 0)),
        out_shape=jax.ShapeDtypeStruct((bsz, h, ssz, dh), _F32),
    )(q, k, v)


def _oproj_body(x_ref, o_ref, w_ref, out_ref):
    out_ref[...] = x_ref[...] + _dot(o_ref[...], w_ref[...])


def _oproj(x2, o2, w_o, bs):
    n, d = x2.shape
    return pl.pallas_call(
        _oproj_body,
        grid=(n // bs,),
        in_specs=[
            pl.BlockSpec((bs, d), lambda i: (i, 0)),
            pl.BlockSpec((bs, d), lambda i: (i, 0)),
            pl.BlockSpec((d, d), lambda i: (0, 0)),
        ],
        out_specs=pl.BlockSpec((bs, d), lambda i: (i, 0)),
        out_shape=jax.ShapeDtypeStruct((n, d), _F32),
    )(x2, o2, w_o)


def _mem_body(h2_ref, sm_ref, kk_ref, kv_ref, x1_ref, out_ref, *, scale):
    hm = _dot(h2_ref[0], sm_ref[0])
    s = _dot_nt(hm, kk_ref[...]) * scale
    m = jnp.max(s, axis=-1, keepdims=True)
    p = jnp.exp(s - m)
    p = p / jnp.sum(p, axis=-1, keepdims=True)
    out_ref[0] = x1_ref[0] + _dot(p, kv_ref[...])


def _mem(h2, shared_m, k_k, k_v, x1, bs, scale):
    bsz, ssz, d = h2.shape
    r = shared_m.shape[-1]
    nk = k_k.shape[0]
    big = pl.BlockSpec((1, bs, d), lambda bi, si: (bi, si, 0))
    return pl.pallas_call(
        functools.partial(_mem_body, scale=scale),
        grid=(bsz, ssz // bs),
        in_specs=[
            big,
            pl.BlockSpec((1, d, r), lambda bi, si: (bi, 0, 0)),
            pl.BlockSpec((nk, r), lambda bi, si: (0, 0)),
            pl.BlockSpec((nk, d), lambda bi, si: (0, 0)),
            big,
        ],
        out_specs=big,
        out_shape=jax.ShapeDtypeStruct((bsz, ssz, d), _F32),
    )(h2, shared_m, k_k, k_v, x1)


def kernel(x, importance, W_compress_router, W_expand_router_Q,
           W_expand_router_K, W_expand_router_V, W_memory_router,
           compress_neurons, expand_neurons_pool, knowledge_K, knowledge_V,
           W_O, g1, b1, g2, b2):
    bsz, ssz, d = x.shape
    n_c, _, r = compress_neurons.shape
    nk = knowledge_K.shape[0]
    h = _N_HEADS
    dh = d // h
    bs = min(512, ssz)

    w_cat = jnp.concatenate([W_compress_router, W_expand_router_Q,
                             W_expand_router_K, W_expand_router_V], axis=1)
    h1, wacc, isum = _ln_route(x, importance, w_cat, g1, b1, 4, bs)
    wsel = _topk(wacc, isum, (16, 8, 8, 8))

    c_flat = compress_neurons.reshape(n_c, d * r)
    e_flat = expand_neurons_pool.reshape(n_c, r * d)
    w_c = wsel[:, :64]
    w_qkv = wsel[:, 64:].reshape(bsz, 3, 64).transpose(1, 0, 2).reshape(
        3 * bsz, 64)
    shared_c = _combine(w_c, c_flat, d * r // 8).reshape(bsz, d, r)
    shared_e = _combine(w_qkv, e_flat, r * d // 8).reshape(3, bsz, r, d)

    q, k, v = _qkv(h1, shared_c, shared_e[0], shared_e[1], shared_e[2], bs)
    q4 = q.reshape(bsz, ssz, h, dh).transpose(0, 2, 1, 3)
    k4 = k.reshape(bsz, ssz, h, dh).transpose(0, 2, 1, 3)
    v4 = v.reshape(bsz, ssz, h, dh).transpose(0, 2, 1, 3)
    o4 = _attn(q4, k4, v4, bs, 1.0 / (dh ** 0.5))
    o = o4.transpose(0, 2, 1, 3).reshape(bsz * ssz, d)
    x1 = _oproj(x.reshape(bsz * ssz, d), o, W_O, bs).reshape(bsz, ssz, d)

    h2, wacc_m, isum_m = _ln_route(x1, importance, W_memory_router, g2, b2,
                                   1, bs)
    wsel_m = _topk(wacc_m, isum_m, (16,))
    shared_m = _combine(wsel_m, c_flat, d * r // 8).reshape(bsz, d, r)
    mbs = min(256, ssz)
    return _mem(h2, shared_m, knowledge_K, knowledge_V, x1, mbs,
                1.0 / (r ** 0.5))


# speedup vs baseline: 1.4021x; 1.4021x over previous
import functools

import jax
import jax.numpy as jnp
from jax.experimental import pallas as pl

_N_HEADS = 16
_F32 = jnp.float32


def _dot(a, b):
    return jnp.dot(a, b, preferred_element_type=_F32)


def _dot_nt(a, b):
    return jax.lax.dot_general(a, b, (((1,), (1,)), ((), ())),
                               preferred_element_type=_F32)


def _ln_route_body(x_ref, imp_ref, w_ref, g_ref, b_ref,
                   h1_ref, wacc_ref, isum_ref, *, n_groups):
    s = pl.program_id(1)
    xb = x_ref[0]
    mu = jnp.mean(xb, axis=-1, keepdims=True)
    xc = xb - mu
    var = jnp.mean(xc * xc, axis=-1, keepdims=True)
    h1 = xc * jax.lax.rsqrt(var + 1e-5) * g_ref[...] + b_ref[...]
    h1_ref[0] = h1
    logits = _dot(h1, w_ref[...])
    ps = []
    for r in range(n_groups):
        lg = logits[:, r * 64:(r + 1) * 64]
        m = jnp.max(lg, axis=-1, keepdims=True)
        e = jnp.exp(lg - m)
        ps.append(e / jnp.sum(e, axis=-1, keepdims=True))
    p = jnp.concatenate(ps, axis=-1) if n_groups > 1 else ps[0]
    imp = imp_ref[0, 0]

    @pl.when(s == 0)
    def _():
        wacc_ref[...] = jnp.zeros_like(wacc_ref)
        isum_ref[...] = jnp.zeros_like(isum_ref)

    wacc_ref[0] += _dot(imp, p)
    isum_ref[0] += jnp.sum(imp)


def _ln_route(x, importance, w_cat, g, b, n_groups, bs):
    bsz, ssz, d = x.shape
    grid = (bsz, ssz // bs)
    imp4 = importance.reshape(bsz, ssz // bs, 1, bs)
    h1, wacc, isum = pl.pallas_call(
        functools.partial(_ln_route_body, n_groups=n_groups),
        grid=grid,
        in_specs=[
            pl.BlockSpec((1, bs, d), lambda bi, si: (bi, si, 0)),
            pl.BlockSpec((1, 1, 1, bs), lambda bi, si: (bi, si, 0, 0)),
            pl.BlockSpec((d, n_groups * 64), lambda bi, si: (0, 0)),
            pl.BlockSpec((1, d), lambda bi, si: (0, 0)),
            pl.BlockSpec((1, d), lambda bi, si: (0, 0)),
        ],
        out_specs=[
            pl.BlockSpec((1, bs, d), lambda bi, si: (bi, si, 0)),
            pl.BlockSpec((1, 1, n_groups * 64), lambda bi, si: (bi, 0, 0)),
            pl.BlockSpec((1, 1, 128), lambda bi, si: (bi, 0, 0)),
        ],
        out_shape=[
            jax.ShapeDtypeStruct((bsz, ssz, d), _F32),
            jax.ShapeDtypeStruct((bsz, 1, n_groups * 64), _F32),
            jax.ShapeDtypeStruct((bsz, 1, 128), _F32),
        ],
    )(x, imp4, w_cat, g.reshape(1, d), b.reshape(1, d))
    return h1, wacc.reshape(bsz, n_groups * 64), isum.reshape(bsz, 128)


def _topk_body(wacc_ref, isum_ref, wsel_ref, *, ks):
    c = isum_ref[:, :1] + 1e-8
    outs = []
    for r, k in enumerate(ks):
        w = wacc_ref[:, r * 64:(r + 1) * 64] / c
        w = w / (jnp.sum(w, axis=-1, keepdims=True) + 1e-8)
        iota = jax.lax.broadcasted_iota(jnp.int32, w.shape, 1)
        vals = w
        sel = jnp.zeros_like(w)
        for _ in range(k):
            m = jnp.max(vals, axis=-1, keepdims=True)
            idx = jnp.min(jnp.where(vals == m, iota, 64), axis=-1,
                          keepdims=True)
            oh = iota == idx
            sel = jnp.where(oh, w, sel)
            vals = jnp.where(oh, -jnp.inf, vals)
        outs.append(sel / (jnp.sum(sel, axis=-1, keepdims=True) + 1e-8))
    wsel_ref[...] = jnp.concatenate(outs, axis=-1) if len(ks) > 1 else outs[0]


def _topk(wacc, isum, ks):
    bsz, nw = wacc.shape
    return pl.pallas_call(
        functools.partial(_topk_body, ks=ks),
        out_shape=jax.ShapeDtypeStruct((bsz, nw), _F32),
    )(wacc, isum)


def _combine_body(w_ref, pool_ref, out_ref):
    out_ref[...] = _dot(w_ref[...], pool_ref[...])


def _combine(wsel, pool_flat, chunk):
    rows, n = wsel.shape
    _, width = pool_flat.shape
    return pl.pallas_call(
        _combine_body,
        grid=(width // chunk,),
        in_specs=[
            pl.BlockSpec((rows, n), lambda ci: (0, 0)),
            pl.BlockSpec((n, chunk), lambda ci: (0, ci)),
        ],
        out_specs=pl.BlockSpec((rows, chunk), lambda ci: (0, ci)),
        out_shape=jax.ShapeDtypeStruct((rows, width), _F32),
    )(wsel, pool_flat)


def _qkv_body(h1_ref, sc_ref, sq_ref, sk_ref, sv_ref, q_ref, k_ref, v_ref):
    h = _dot(h1_ref[0], sc_ref[0])
    q_ref[0] = _dot(h, sq_ref[0])
    k_ref[0] = _dot(h, sk_ref[0])
    v_ref[0] = _dot(h, sv_ref[0])


def _qkv(h1, shared_c, sq, sk, sv, bs):
    bsz, ssz, d = h1.shape
    r = shared_c.shape[-1]
    big = pl.BlockSpec((1, bs, d), lambda bi, si: (bi, si, 0))
    mat_dr = pl.BlockSpec((1, d, r), lambda bi, si: (bi, 0, 0))
    mat_rd = pl.BlockSpec((1, r, d), lambda bi, si: (bi, 0, 0))
    out = jax.ShapeDtypeStruct((bsz, ssz, d), _F32)
    return pl.pallas_call(
        _qkv_body,
        grid=(bsz, ssz // bs),
        in_specs=[big, mat_dr, mat_rd, mat_rd, mat_rd],
        out_specs=[big, big, big],
        out_shape=[out, out, out],
    )(h1, shared_c, sq, sk, sv)


def _attn_body(q_ref, k_ref, v_ref, x_ref, wo_ref, out_ref, *, scale,
               n_heads):
    q = q_ref[0]
    k = k_ref[0]
    v = v_ref[0]
    dh = q.shape[-1] // n_heads
    outs = []
    for hh in range(n_heads):
        sl = slice(hh * dh, (hh + 1) * dh)
        s = _dot_nt(q[:, sl], k[:, sl]) * scale
        m = jnp.max(s, axis=-1, keepdims=True)
        p = jnp.exp(s - m)
        p = p / jnp.sum(p, axis=-1, keepdims=True)
        outs.append(_dot(p, v[:, sl]))
    o = jnp.concatenate(outs, axis=-1)
    out_ref[0] = x_ref[0] + _dot(o, wo_ref[...])


def _attn(q, k, v, x, w_o, bq, scale, n_heads):
    bsz, ssz, d = q.shape
    blk = pl.BlockSpec((1, bq, d), lambda bi, qi: (bi, qi, 0))
    full = pl.BlockSpec((1, ssz, d), lambda bi, qi: (bi, 0, 0))
    return pl.pallas_call(
        functools.partial(_attn_body, scale=scale, n_heads=n_heads),
        grid=(bsz, ssz // bq),
        in_specs=[blk, full, full, blk,
                  pl.BlockSpec((d, d), lambda bi, qi: (0, 0))],
        out_specs=blk,
        out_shape=jax.ShapeDtypeStruct((bsz, ssz, d), _F32),
    )(q, k, v, x, w_o)


def _mem_body(h2_ref, sm_ref, kk_ref, kv_ref, x1_ref, out_ref, *, scale):
    hm = _dot(h2_ref[0], sm_ref[0])
    s = _dot_nt(hm, kk_ref[...]) * scale
    m = jnp.max(s, axis=-1, keepdims=True)
    p = jnp.exp(s - m)
    p = p / jnp.sum(p, axis=-1, keepdims=True)
    out_ref[0] = x1_ref[0] + _dot(p, kv_ref[...])


def _mem(h2, shared_m, k_k, k_v, x1, bs, scale):
    bsz, ssz, d = h2.shape
    r = shared_m.shape[-1]
    nk = k_k.shape[0]
    big = pl.BlockSpec((1, bs, d), lambda bi, si: (bi, si, 0))
    return pl.pallas_call(
        functools.partial(_mem_body, scale=scale),
        grid=(bsz, ssz // bs),
        in_specs=[
            big,
            pl.BlockSpec((1, d, r), lambda bi, si: (bi, 0, 0)),
            pl.BlockSpec((nk, r), lambda bi, si: (0, 0)),
            pl.BlockSpec((nk, d), lambda bi, si: (0, 0)),
            big,
        ],
        out_specs=big,
        out_shape=jax.ShapeDtypeStruct((bsz, ssz, d), _F32),
    )(h2, shared_m, k_k, k_v, x1)


def kernel(x, importance, W_compress_router, W_expand_router_Q,
           W_expand_router_K, W_expand_router_V, W_memory_router,
           compress_neurons, expand_neurons_pool, knowledge_K, knowledge_V,
           W_O, g1, b1, g2, b2):
    bsz, ssz, d = x.shape
    n_c, _, r = compress_neurons.shape
    nk = knowledge_K.shape[0]
    h = _N_HEADS
    dh = d // h
    bs = min(512, ssz)

    w_cat = jnp.concatenate([W_compress_router, W_expand_router_Q,
                             W_expand_router_K, W_expand_router_V], axis=1)
    h1, wacc, isum = _ln_route(x, importance, w_cat, g1, b1, 4, bs)
    wsel = _topk(wacc, isum, (16, 8, 8, 8))

    c_flat = compress_neurons.reshape(n_c, d * r)
    e_flat = expand_neurons_pool.reshape(n_c, r * d)
    w_c = wsel[:, :64]
    w_qkv = wsel[:, 64:].reshape(bsz, 3, 64).transpose(1, 0, 2).reshape(
        3 * bsz, 64)
    shared_c = _combine(w_c, c_flat, d * r // 8).reshape(bsz, d, r)
    shared_e = _combine(w_qkv, e_flat, r * d // 8).reshape(3, bsz, r, d)

    q, k, v = _qkv(h1, shared_c, shared_e[0], shared_e[1], shared_e[2], bs)
    x1 = _attn(q, k, v, x, W_O, bs, 1.0 / (dh ** 0.5), h)

    h2, wacc_m, isum_m = _ln_route(x1, importance, W_memory_router, g2, b2,
                                   1, bs)
    wsel_m = _topk(wacc_m, isum_m, (16,))
    shared_m = _combine(wsel_m, c_flat, d * r // 8).reshape(bsz, d, r)
    mbs = min(256, ssz)
    return _mem(h2, shared_m, knowledge_K, knowledge_V, x1, mbs,
                1.0 / (r ** 0.5))
